# Initial kernel scaffold; baseline (speedup 1.0000x reference)
import jax
import jax.numpy as jnp
from jax.experimental import pallas as pl


def kernel(img, params):
    raise NotImplementedError("write your pallas kernel here")


# speedup vs baseline: 2.4542x; 2.4542x over previous
import jax
import jax.numpy as jnp
import numpy as np
from jax.experimental import pallas as pl

DIM = 384
PATCH = 14
IMG = 224
NCLS = 1000
LH = 4
LD = 32
WS = 8
HH = 8
HD = 64
KQ = 64
KKV = 64
KFF = 64
LMULT = 2
HMULT = 4
PDIM = 3 * PATCH * PATCH
GRID = IMG // PATCH
NW = GRID // WS
N = GRID * GRID
B = 8


def _posemb_np(h, w, dim):
    y, x = np.meshgrid(np.arange(h), np.arange(w), indexing='ij')
    omega = np.arange(dim // 4) / (dim // 4 - 1)
    omega = 1.0 / (10000.0 ** omega)
    y = y.reshape(-1)[:, None] * omega[None, :]
    x = x.reshape(-1)[:, None] * omega[None, :]
    pe = np.concatenate((np.sin(x), np.cos(x), np.sin(y), np.cos(y)), axis=1)
    return pe.reshape(h * w, dim).astype(np.float32)


def _ln(x, g, b):
    m = x.mean(-1, keepdims=True)
    v = ((x - m) ** 2).mean(-1, keepdims=True)
    return (x - m) * jax.lax.rsqrt(v + 1e-5) * g + b


def _topk_oh(s, k):
    r, n = s.shape
    iota = jax.lax.broadcasted_iota(jnp.int32, (r, n), 1)
    m = s
    rows = []
    for _ in range(k):
        idx = jnp.argmax(m, axis=-1).astype(jnp.int32)[:, None]
        hit = iota == idx
        rows.append(hit.astype(jnp.float32)[:, None, :])
        m = jnp.where(hit, -jnp.inf, m)
    return jnp.concatenate(rows, axis=1)


def _window(x):
    xw = x.reshape(B, NW, WS, NW, WS, DIM).transpose(0, 1, 3, 2, 4, 5)
    return xw.reshape(B * NW * NW, WS * WS, DIM)


def _unwindow(xw):
    x = xw.reshape(B, NW, NW, WS, WS, DIM).transpose(0, 1, 3, 2, 4, 5)
    return x.reshape(B, N, DIM)


def _heads(t, nh, hd):
    nb = t.shape[0]
    tok = t.shape[1]
    return t.reshape(nb, tok, nh, hd).transpose(0, 2, 1, 3).reshape(nb * nh, tok, hd)


def _unheads(t, nb, nh, tok, hd):
    return t.reshape(nb, nh, tok, hd).transpose(0, 2, 1, 3).reshape(nb, tok, nh * hd)


def _attn(q, k, v, scale):
    s = jnp.einsum('bid,bjd->bij', q, k, preferred_element_type=jnp.float32) * scale
    s = s - s.max(-1, keepdims=True)
    e = jnp.exp(s)
    a = e / e.sum(-1, keepdims=True)
    return jnp.einsum('bij,bjd->bid', a, v, preferred_element_type=jnp.float32)


def _mm(x, w):
    return jnp.dot(x, w, preferred_element_type=jnp.float32)


def _body(refs):
    it = iter(refs[:-1])
    nxt = lambda: next(it)[...]
    out_ref = refs[-1]

    xp = nxt()
    posemb = nxt()
    pe_ln0_g = nxt(); pe_ln0_b = nxt()
    pe_w = nxt(); pe_b = nxt()
    pe_ln1_g = nxt(); pe_ln1_b = nxt()

    xf = _ln(xp.reshape(B * N, PDIM), pe_ln0_g, pe_ln0_b)
    xf = _mm(xf, pe_w) + pe_b
    xf = _ln(xf, pe_ln1_g, pe_ln1_b)
    x = xf.reshape(B, N, DIM) + posemb[None]

    for _ in range(2):
        a_ln_l_g = nxt(); a_ln_l_b = nxt()
        qkv_l = nxt(); out_l = nxt(); out_l_b = nxt()
        a_ln_h_g = nxt(); a_ln_h_b = nxt()
        r2 = nxt()
        w_q = nxt(); w_k = nxt(); w_v = nxt()
        out_h = nxt(); out_h_b = nxt(); null_out = nxt()

        xl = _ln(x, a_ln_l_g, a_ln_l_b)
        xw = _window(xl)
        nwin = xw.shape[0]
        qkv = _mm(xw.reshape(nwin * WS * WS, DIM), qkv_l).reshape(nwin, WS * WS, 3 * LH * LD)
        q = _heads(qkv[:, :, : LH * LD], LH, LD)
        k = _heads(qkv[:, :, LH * LD: 2 * LH * LD], LH, LD)
        v = _heads(qkv[:, :, 2 * LH * LD:], LH, LD)
        o = _attn(q, k, v, LD ** -0.5)
        o = _unheads(o, nwin, LH, WS * WS, LD)
        o = _mm(o.reshape(nwin * WS * WS, LH * LD), out_l) + out_l_b
        light = _unwindow(o.reshape(nwin, WS * WS, DIM))

        xn = _ln(x, a_ln_h_g, a_ln_h_b)
        s2 = _mm(xn.reshape(B * N, DIM), r2).reshape(B, N, 2)
        st = s2.transpose(0, 2, 1).reshape(B * 2, N)
        oh = _topk_oh(st, KQ).reshape(B, 2, KQ, N)
        ohq = oh[:, 0]
        ohkv = oh[:, 1]
        qt = jnp.einsum('bkn,bnd->bkd', ohq, xn, preferred_element_type=jnp.float32)
        kvt = jnp.einsum('bkn,bnd->bkd', ohkv, xn, preferred_element_type=jnp.float32)
        q = _heads(_mm(qt.reshape(B * KQ, DIM), w_q).reshape(B, KQ, HH * HD), HH, HD)
        k = _heads(_mm(kvt.reshape(B * KKV, DIM), w_k).reshape(B, KKV, HH * HD), HH, HD)
        v = _heads(_mm(kvt.reshape(B * KKV, DIM), w_v).reshape(B, KKV, HH * HD), HH, HD)
        o = _attn(q, k, v, HD ** -0.5)
        o = _unheads(o, B, HH, KQ, HD)
        o = (_mm(o.reshape(B * KQ, HH * HD), out_h) + out_h_b).reshape(B, KQ, DIM)
        covered = ohq.sum(axis=1)[:, :, None]
        heavy = jnp.einsum('bkn,bkd->bnd', ohq, o, preferred_element_type=jnp.float32)
        heavy = heavy + (1.0 - covered) * null_out[None]
        x = x + light + heavy

        f_ln_l_g = nxt(); f_ln_l_b = nxt()
        w1_l = nxt(); b1_l = nxt(); w2_l = nxt(); b2_l = nxt()
        f_ln_h_g = nxt(); f_ln_h_b = nxt()
        r_w = nxt()
        w1_h = nxt(); b1_h = nxt(); w2_h = nxt(); b2_h = nxt()

        xl = _ln(x, f_ln_l_g, f_ln_l_b).reshape(B * N, DIM)
        light = _mm(jax.nn.gelu(_mm(xl, w1_l) + b1_l), w2_l) + b2_l
        light = light.reshape(B, N, DIM)

        xn = _ln(x, f_ln_h_g, f_ln_h_b)
        s = _mm(xn.reshape(B * N, DIM), r_w).reshape(B, N).reshape(B, 1, N).reshape(B, N)
        oh = _topk_oh(s, KFF)
        t = jnp.einsum('bkn,bnd->bkd', oh, xn, preferred_element_type=jnp.float32)
        hvy = _mm(jax.nn.gelu(_mm(t.reshape(B * KFF, DIM), w1_h) + b1_h), w2_h) + b2_h
        hvy = hvy.reshape(B, KFF, DIM)
        scattered = jnp.einsum('bkn,bkd->bnd', oh, hvy, preferred_element_type=jnp.float32)
        x = x + light + scattered

    hd_ln_g = nxt(); hd_ln_b = nxt()
    hd_w = nxt(); hd_b = nxt()
    pooled = x.mean(axis=1)
    pooled = _ln(pooled, hd_ln_g, hd_ln_b)
    out_ref[...] = _mm(pooled, hd_w) + hd_b


def kernel(img, params):
    b, c, H, W = img.shape
    h, w = H // PATCH, W // PATCH
    xp = img.reshape(b, c, h, PATCH, w, PATCH).transpose(0, 2, 4, 3, 5, 1).reshape(b, h * w, PDIM)
    posemb = jnp.asarray(_posemb_np(h, w, DIM))

    row = lambda a: a.reshape(1, -1)
    inputs = [xp, posemb,
              row(params['pe_ln0_g']), row(params['pe_ln0_b']),
              params['pe_w'], row(params['pe_b']),
              row(params['pe_ln1_g']), row(params['pe_ln1_b'])]
    for lp in params['layers']:
        a = lp['attn']
        inputs += [row(a['ln_l_g']), row(a['ln_l_b']),
                   a['qkv_l'], a['out_l'], row(a['out_l_b']),
                   row(a['ln_h_g']), row(a['ln_h_b']),
                   jnp.stack([a['r_q'], a['r_kv']], axis=1),
                   a['w_q'], a['w_k'], a['w_v'],
                   a['out_h'], row(a['out_h_b']), row(a['null_out'])]
        f = lp['ff']
        inputs += [row(f['ln_l_g']), row(f['ln_l_b']),
                   f['w1_l'], row(f['b1_l']), f['w2_l'], row(f['b2_l']),
                   row(f['ln_h_g']), row(f['ln_h_b']),
                   f['r_w'].reshape(-1, 1),
                   f['w1_h'], row(f['b1_h']), f['w2_h'], row(f['b2_h'])]
    inputs += [row(params['hd_ln_g']), row(params['hd_ln_b']),
               params['hd_w'], row(params['hd_b'])]

    return pl.pallas_call(
        lambda *refs: _body(refs),
        out_shape=jax.ShapeDtypeStruct((b, NCLS), jnp.float32),
    )(*inputs)


# speedup vs baseline: 2.7222x; 1.1092x over previous
import jax
import jax.numpy as jnp
import numpy as np
from jax.experimental import pallas as pl

DIM = 384
PATCH = 14
IMG = 224
NCLS = 1000
LH = 4
LD = 32
WS = 8
HH = 8
HD = 64
KQ = 64
KKV = 64
KFF = 64
LMULT = 2
HMULT = 4
PDIM = 3 * PATCH * PATCH
GRID = IMG // PATCH
NW = GRID // WS
N = GRID * GRID
B = 8


def _posemb_np(h, w, dim):
    y, x = np.meshgrid(np.arange(h), np.arange(w), indexing='ij')
    omega = np.arange(dim // 4) / (dim // 4 - 1)
    omega = 1.0 / (10000.0 ** omega)
    y = y.reshape(-1)[:, None] * omega[None, :]
    x = x.reshape(-1)[:, None] * omega[None, :]
    pe = np.concatenate((np.sin(x), np.cos(x), np.sin(y), np.cos(y)), axis=1)
    return pe.reshape(h * w, dim).astype(np.float32)


def _ln(x, g, b):
    m = x.mean(-1, keepdims=True)
    v = ((x - m) ** 2).mean(-1, keepdims=True)
    return (x - m) * jax.lax.rsqrt(v + 1e-5) * g + b


def _topk_oh(s, k):
    r, n = s.shape
    ii = jax.lax.broadcasted_iota(jnp.int32, (n, n), 0)
    jj = jax.lax.broadcasted_iota(jnp.int32, (n, n), 1)
    C = 8
    si = s[:, None, :]
    rank = jnp.zeros((r, n), jnp.float32)
    for c0 in range(0, n, C):
        sjc = s[:, c0:c0 + C, None]
        jidx = jax.lax.broadcasted_iota(jnp.int32, (C, n), 0) + c0
        iidx = jax.lax.broadcasted_iota(jnp.int32, (C, n), 1)
        jlt = (jidx < iidx)[None]
        cmp = jnp.where((sjc > si) | ((sjc == si) & jlt), 1.0, 0.0)
        rank = rank + jnp.sum(cmp, axis=1)
    sel = rank < k
    lt = jnp.where(jj < ii, 1.0, 0.0)
    pos = jax.lax.dot_general(sel.astype(jnp.float32), lt,
                              (((1,), (1,)), ((), ())),
                              preferred_element_type=jnp.float32)
    kio = jax.lax.broadcasted_iota(jnp.int32, (r, k, n), 1)
    posi = pos.astype(jnp.int32)
    return jnp.where(sel[:, None, :] & (posi[:, None, :] == kio), 1.0, 0.0)


def _window(x):
    xw = x.reshape(B, NW, WS, NW, WS, DIM).transpose(0, 1, 3, 2, 4, 5)
    return xw.reshape(B * NW * NW, WS * WS, DIM)


def _unwindow(xw):
    x = xw.reshape(B, NW, NW, WS, WS, DIM).transpose(0, 1, 3, 2, 4, 5)
    return x.reshape(B, N, DIM)


def _heads(t, nh, hd):
    nb = t.shape[0]
    tok = t.shape[1]
    return t.reshape(nb, tok, nh, hd).transpose(0, 2, 1, 3).reshape(nb * nh, tok, hd)


def _unheads(t, nb, nh, tok, hd):
    return t.reshape(nb, nh, tok, hd).transpose(0, 2, 1, 3).reshape(nb, tok, nh * hd)


def _attn(q, k, v, scale):
    s = jnp.einsum('bid,bjd->bij', q, k, preferred_element_type=jnp.float32) * scale
    s = s - s.max(-1, keepdims=True)
    e = jnp.exp(s)
    a = e / e.sum(-1, keepdims=True)
    return jnp.einsum('bij,bjd->bid', a, v, preferred_element_type=jnp.float32)


def _mm(x, w):
    return jnp.dot(x, w, preferred_element_type=jnp.float32)


def _body(refs):
    it = iter(refs[:-1])
    nxt = lambda: next(it)[...]
    out_ref = refs[-1]

    xp = nxt()
    posemb = nxt()
    pe_ln0_g = nxt(); pe_ln0_b = nxt()
    pe_w = nxt(); pe_b = nxt()
    pe_ln1_g = nxt(); pe_ln1_b = nxt()

    xf = _ln(xp.reshape(B * N, PDIM), pe_ln0_g, pe_ln0_b)
    xf = _mm(xf, pe_w) + pe_b
    xf = _ln(xf, pe_ln1_g, pe_ln1_b)
    x = xf.reshape(B, N, DIM) + posemb[None]

    for _ in range(2):
        a_ln_l_g = nxt(); a_ln_l_b = nxt()
        qkv_l = nxt(); out_l = nxt(); out_l_b = nxt()
        a_ln_h_g = nxt(); a_ln_h_b = nxt()
        r2 = nxt()
        w_q = nxt(); w_k = nxt(); w_v = nxt()
        out_h = nxt(); out_h_b = nxt(); null_out = nxt()

        xl = _ln(x, a_ln_l_g, a_ln_l_b)
        xw = _window(xl)
        nwin = xw.shape[0]
        qkv = _mm(xw.reshape(nwin * WS * WS, DIM), qkv_l).reshape(nwin, WS * WS, 3 * LH * LD)
        q = _heads(qkv[:, :, : LH * LD], LH, LD)
        k = _heads(qkv[:, :, LH * LD: 2 * LH * LD], LH, LD)
        v = _heads(qkv[:, :, 2 * LH * LD:], LH, LD)
        o = _attn(q, k, v, LD ** -0.5)
        o = _unheads(o, nwin, LH, WS * WS, LD)
        o = _mm(o.reshape(nwin * WS * WS, LH * LD), out_l) + out_l_b
        light = _unwindow(o.reshape(nwin, WS * WS, DIM))

        xn = _ln(x, a_ln_h_g, a_ln_h_b)
        s2 = _mm(xn.reshape(B * N, DIM), r2).reshape(B, N, 2)
        st = s2.transpose(0, 2, 1).reshape(B * 2, N)
        oh = _topk_oh(st, KQ).reshape(B, 2, KQ, N)
        ohq = oh[:, 0]
        ohkv = oh[:, 1]
        qt = jnp.einsum('bkn,bnd->bkd', ohq, xn, preferred_element_type=jnp.float32)
        kvt = jnp.einsum('bkn,bnd->bkd', ohkv, xn, preferred_element_type=jnp.float32)
        q = _heads(_mm(qt.reshape(B * KQ, DIM), w_q).reshape(B, KQ, HH * HD), HH, HD)
        k = _heads(_mm(kvt.reshape(B * KKV, DIM), w_k).reshape(B, KKV, HH * HD), HH, HD)
        v = _heads(_mm(kvt.reshape(B * KKV, DIM), w_v).reshape(B, KKV, HH * HD), HH, HD)
        o = _attn(q, k, v, HD ** -0.5)
        o = _unheads(o, B, HH, KQ, HD)
        o = (_mm(o.reshape(B * KQ, HH * HD), out_h) + out_h_b).reshape(B, KQ, DIM)
        covered = ohq.sum(axis=1)[:, :, None]
        heavy = jnp.einsum('bkn,bkd->bnd', ohq, o, preferred_element_type=jnp.float32)
        heavy = heavy + (1.0 - covered) * null_out[None]
        x = x + light + heavy

        f_ln_l_g = nxt(); f_ln_l_b = nxt()
        w1_l = nxt(); b1_l = nxt(); w2_l = nxt(); b2_l = nxt()
        f_ln_h_g = nxt(); f_ln_h_b = nxt()
        r_w = nxt()
        w1_h = nxt(); b1_h = nxt(); w2_h = nxt(); b2_h = nxt()

        xl = _ln(x, f_ln_l_g, f_ln_l_b).reshape(B * N, DIM)
        light = _mm(jax.nn.gelu(_mm(xl, w1_l) + b1_l), w2_l) + b2_l
        light = light.reshape(B, N, DIM)

        xn = _ln(x, f_ln_h_g, f_ln_h_b)
        s = _mm(xn.reshape(B * N, DIM), r_w).reshape(B, N).reshape(B, 1, N).reshape(B, N)
        oh = _topk_oh(s, KFF)
        t = jnp.einsum('bkn,bnd->bkd', oh, xn, preferred_element_type=jnp.float32)
        hvy = _mm(jax.nn.gelu(_mm(t.reshape(B * KFF, DIM), w1_h) + b1_h), w2_h) + b2_h
        hvy = hvy.reshape(B, KFF, DIM)
        scattered = jnp.einsum('bkn,bkd->bnd', oh, hvy, preferred_element_type=jnp.float32)
        x = x + light + scattered

    hd_ln_g = nxt(); hd_ln_b = nxt()
    hd_w = nxt(); hd_b = nxt()
    pooled = x.mean(axis=1)
    pooled = _ln(pooled, hd_ln_g, hd_ln_b)
    out_ref[...] = _mm(pooled, hd_w) + hd_b


def kernel(img, params):
    b, c, H, W = img.shape
    h, w = H // PATCH, W // PATCH
    xp = img.reshape(b, c, h, PATCH, w, PATCH).transpose(0, 2, 4, 3, 5, 1).reshape(b, h * w, PDIM)
    posemb = jnp.asarray(_posemb_np(h, w, DIM))

    row = lambda a: a.reshape(1, -1)
    inputs = [xp, posemb,
              row(params['pe_ln0_g']), row(params['pe_ln0_b']),
              params['pe_w'], row(params['pe_b']),
              row(params['pe_ln1_g']), row(params['pe_ln1_b'])]
    for lp in params['layers']:
        a = lp['attn']
        inputs += [row(a['ln_l_g']), row(a['ln_l_b']),
                   a['qkv_l'], a['out_l'], row(a['out_l_b']),
                   row(a['ln_h_g']), row(a['ln_h_b']),
                   jnp.stack([a['r_q'], a['r_kv']], axis=1),
                   a['w_q'], a['w_k'], a['w_v'],
                   a['out_h'], row(a['out_h_b']), row(a['null_out'])]
        f = lp['ff']
        inputs += [row(f['ln_l_g']), row(f['ln_l_b']),
                   f['w1_l'], row(f['b1_l']), f['w2_l'], row(f['b2_l']),
                   row(f['ln_h_g']), row(f['ln_h_b']),
                   f['r_w'].reshape(-1, 1),
                   f['w1_h'], row(f['b1_h']), f['w2_h'], row(f['b2_h'])]
    inputs += [row(params['hd_ln_g']), row(params['hd_ln_b']),
               params['hd_w'], row(params['hd_b'])]

    return pl.pallas_call(
        lambda *refs: _body(refs),
        out_shape=jax.ShapeDtypeStruct((b, NCLS), jnp.float32),
    )(*inputs)
